# baseline (speedup 1.0000x reference)
import functools

import jax
import jax.numpy as jnp
from jax import lax
from jax.experimental import pallas as pl
from jax.experimental.pallas import tpu as pltpu
from jax.experimental.pallas import tpu_sc as plsc

_CUTOFF = 5.0
_K = 8
_E = 3_200_000
_N = 100_000
_NC = 2
_NS = 16
_NW = _NC * _NS
_C = 1024
_CHUNKS = _E // _C
_RPT = (_N // _NS) // 8 * 8
_REM = _N - _NS * _RPT
_GROUPS = _C // 16
_BN = 10_000


def _gram_body(a_ref, b_ref, f_ref, ftf_ref, cs_ref):
    fb = a_ref[...] + b_ref[...]
    f_ref[...] = fb

    @pl.when(pl.program_id(0) == 0)
    def _():
        ftf_ref[...] = jnp.zeros((_K, _K), jnp.float32)
        cs_ref[...] = jnp.zeros((_K, _K), jnp.float32)

    ftf_ref[...] += lax.dot_general(
        fb, fb, (((0,), (0,)), ((), ())), preferred_element_type=jnp.float32
    )
    cs_ref[...] += jnp.broadcast_to(
        jnp.sum(fb, axis=0, keepdims=True), (_K, _K)
    )


_gram_kernel = pl.pallas_call(
    _gram_body,
    grid=(_N // _BN,),
    in_specs=[
        pl.BlockSpec((_BN, _K), lambda i: (i, 0)),
        pl.BlockSpec((_BN, _K), lambda i: (i + _N // _BN, 0)),
    ],
    out_specs=[
        pl.BlockSpec((_BN, _K), lambda i: (i, 0)),
        pl.BlockSpec((_K, _K), lambda i: (0, 0)),
        pl.BlockSpec((_K, _K), lambda i: (0, 0)),
    ],
    out_shape=[
        jax.ShapeDtypeStruct((_N, _K), jnp.float32),
        jax.ShapeDtypeStruct((_K, _K), jnp.float32),
        jax.ShapeDtypeStruct((_K, _K), jnp.float32),
    ],
)


def _seg_body(
    r_hbm, fa_hbm, out_hbm, rb, idxb, rows, zbuf, facc, sem_r, sem_i, sem_sc
):
    c = lax.axis_index("c")
    s = lax.axis_index("s")
    w = s * _NC + c

    ii = lax.iota(jnp.int32, 16)
    ones = jnp.full((16,), 1.0, jnp.float32)
    zeros = jnp.zeros((16,), jnp.float32)

    def zero_body(t, _):
        f = t * 16 + ii
        plsc.store_scatter(zbuf, [f >> 3, f & 7], zeros)
        return ()

    lax.fori_loop(0, (_RPT + _REM) * _K // 16, zero_body, ())

    def ones_body(t, _):
        plsc.store_scatter(
            rows, [t * 16 + ii, jnp.full((16,), 0, jnp.int32)], ones
        )
        return ()

    lax.fori_loop(0, 2 * _C // 16, ones_body, ())
    pltpu.sync_copy(zbuf.at[pl.ds(0, _RPT)], facc.at[pl.ds(s * _RPT, _RPT)])

    @pl.when(s == _NS - 1)
    def _():
        pltpu.sync_copy(
            zbuf.at[pl.ds(0, _REM)], facc.at[pl.ds(_NS * _RPT, _REM)]
        )

    plsc.subcore_barrier()

    n_chunks = jnp.where(w < _CHUNKS % _NW, _CHUNKS // _NW + 1, _CHUNKS // _NW)

    def issue_inputs(t):
        m = w + t * _NW
        off = (t & 1) * _C
        pltpu.async_copy(
            r_hbm.at[pl.ds(m * _C, _C)], rb.at[pl.ds(off, _C)], sem_r
        )
        pltpu.async_copy(
            fa_hbm.at[pl.ds(m * (_C // 128), _C // 128)],
            idxb.at[pl.ds((t & 1) * (_C // 128), _C // 128)],
            sem_i,
        )

    issue_inputs(0)

    def chunk_body(t, _):
        off = (t & 1) * _C
        ioff = (t & 1) * (_C // 128)

        @pl.when(t > 0)
        def _():
            poff = ((t - 1) & 1) * _C
            pltpu.make_async_copy(
                out_hbm.at[pl.ds(0, _C)], rows.at[pl.ds(poff, _C)], sem_sc
            ).wait()

        @pl.when(t + 1 < n_chunks)
        def _():
            issue_inputs(t + 1)

        pltpu.make_async_copy(
            r_hbm.at[pl.ds(0, _C)], rb.at[pl.ds(off, _C)], sem_r
        ).wait()
        pltpu.make_async_copy(
            fa_hbm.at[pl.ds(0, _C // 128)],
            idxb.at[pl.ds(ioff, _C // 128)],
            sem_i,
        ).wait()

        def group_body(g, _):
            erow = off + g * 16 + ii
            ss = rb[pl.ds(off + g * 16, 16)]
            ssc = jnp.maximum(ss, jnp.float32(1e-37))
            t32 = plsc.bitcast(ssc, jnp.int32)
            t32 = jnp.int32(0x5F3759DF) - (t32 >> 1)
            q = plsc.bitcast(t32, jnp.float32)
            h = ssc * jnp.float32(0.5)
            q = q * (jnp.float32(1.5) - h * q * q)
            q = q * (jnp.float32(1.5) - h * q * q)
            q = q * (jnp.float32(1.5) - h * q * q)
            r = ss * q * jnp.float32(1.0 / _CUTOFF)
            pk = r
            for k in range(1, _K):
                plsc.store_scatter(
                    rows, [erow, jnp.full((16,), k, jnp.int32)], pk
                )
                pk = pk * r
            return ()

        lax.fori_loop(0, _GROUPS, group_body, ())

        for j in range(_C // 128):
            pltpu.async_copy(
                rows.at[pl.ds(off + j * 128, 128)],
                facc.at[idxb.at[ioff + j]],
                sem_sc,
                add=True,
            )
        return ()

    lax.fori_loop(0, n_chunks, chunk_body, ())

    @pl.when(n_chunks > 0)
    def _():
        loff = ((n_chunks - 1) & 1) * _C
        pltpu.make_async_copy(
            out_hbm.at[pl.ds(0, _C)], rows.at[pl.ds(loff, _C)], sem_sc
        ).wait()

    plsc.subcore_barrier()
    r0 = c * _N + s * _RPT
    pltpu.sync_copy(facc.at[pl.ds(s * _RPT, _RPT)], zbuf.at[pl.ds(0, _RPT)])
    pltpu.sync_copy(zbuf.at[pl.ds(0, _RPT)], out_hbm.at[pl.ds(r0, _RPT)])

    @pl.when(s == _NS - 1)
    def _():
        pltpu.sync_copy(
            facc.at[pl.ds(_NS * _RPT, _REM)], zbuf.at[pl.ds(_RPT, _REM)]
        )
        pltpu.sync_copy(
            zbuf.at[pl.ds(_RPT, _REM)],
            out_hbm.at[pl.ds(c * _N + _NS * _RPT, _REM)],
        )


_seg_kernel = functools.partial(
    pl.kernel,
    out_type=jax.ShapeDtypeStruct((_NC * _N, _K), jnp.float32),
    mesh=plsc.VectorSubcoreMesh(core_axis_name="c", subcore_axis_name="s"),
    compiler_params=pltpu.CompilerParams(
        needs_layout_passes=False, use_tc_tiling_on_sc=False
    ),
    scratch_types=[
        pltpu.VMEM((2 * _C,), jnp.float32),
        pltpu.VMEM((2 * (_C // 128), 128), jnp.int32),
        pltpu.VMEM((2 * _C, _K), jnp.float32),
        pltpu.VMEM((_RPT + _REM, _K), jnp.float32),
        pltpu.VMEM_SHARED((_N, _K), jnp.float32),
        pltpu.SemaphoreType.DMA,
        pltpu.SemaphoreType.DMA,
        pltpu.SemaphoreType.DMA,
    ],
)(_seg_body)


def kernel(positions, neighbor_vectors, first_atom):
    n = positions.shape[0]
    ss = jnp.sum(neighbor_vectors * neighbor_vectors, axis=1)
    fa2d = first_atom.reshape(-1, 128)
    parts = _seg_kernel(ss, fa2d)
    features, ftf, cs = _gram_kernel(parts, parts)
    s = cs[0]
    mu = jnp.sum(s) / jnp.float32(n * _K)
    one = jnp.ones((_K,), jnp.float32)
    gram = (
        ftf
        - mu * (jnp.outer(s, one) + jnp.outer(one, s))
        + (n * mu * mu) * jnp.ones((_K, _K), jnp.float32)
    )
    gram = gram + (jnp.float32(1e-6) * jnp.trace(gram)) * jnp.eye(
        _K, dtype=jnp.float32
    )
    a8 = jnp.linalg.cholesky(gram).T
    _, _, vh = jnp.linalg.svd(a8, full_matrices=False)
    return features @ vh[:3].T


# speedup vs baseline: 1.0090x; 1.0090x over previous
import functools

import jax
import jax.numpy as jnp
from jax import lax
from jax._src.tpu.linalg import qdwh as _tpu_qdwh
from jax.experimental import pallas as pl
from jax.experimental.pallas import tpu as pltpu
from jax.experimental.pallas import tpu_sc as plsc

_CUTOFF = 5.0
_K = 8
_E = 3_200_000
_N = 100_000
_NC = 2
_NS = 16
_NW = _NC * _NS
_C = 1024
_CHUNKS = _E // _C
_RPT = (_N // _NS) // 8 * 8
_REM = _N - _NS * _RPT
_GROUPS = _C // 16
_BN = 10_000


def _gram_body(a_ref, b_ref, f_ref, ftf_ref, cs_ref):
    fb = a_ref[...] + b_ref[...]
    f_ref[...] = fb

    @pl.when(pl.program_id(0) == 0)
    def _():
        ftf_ref[...] = jnp.zeros((_K, _K), jnp.float32)
        cs_ref[...] = jnp.zeros((_K, _K), jnp.float32)

    ftf_ref[...] += lax.dot_general(
        fb, fb, (((0,), (0,)), ((), ())), preferred_element_type=jnp.float32
    )
    cs_ref[...] += jnp.broadcast_to(
        jnp.sum(fb, axis=0, keepdims=True), (_K, _K)
    )


_gram_kernel = pl.pallas_call(
    _gram_body,
    grid=(_N // _BN,),
    in_specs=[
        pl.BlockSpec((_BN, _K), lambda i: (i, 0)),
        pl.BlockSpec((_BN, _K), lambda i: (i + _N // _BN, 0)),
    ],
    out_specs=[
        pl.BlockSpec((_BN, _K), lambda i: (i, 0)),
        pl.BlockSpec((_K, _K), lambda i: (0, 0)),
        pl.BlockSpec((_K, _K), lambda i: (0, 0)),
    ],
    out_shape=[
        jax.ShapeDtypeStruct((_N, _K), jnp.float32),
        jax.ShapeDtypeStruct((_K, _K), jnp.float32),
        jax.ShapeDtypeStruct((_K, _K), jnp.float32),
    ],
)


def _seg_body(
    r_hbm, fa_hbm, out_hbm, rb, idxb, rows, zbuf, facc, sem_r, sem_i, sem_sc
):
    c = lax.axis_index("c")
    s = lax.axis_index("s")
    w = s * _NC + c

    ii = lax.iota(jnp.int32, 16)
    ones = jnp.full((16,), 1.0, jnp.float32)
    zeros = jnp.zeros((16,), jnp.float32)

    def zero_body(t, _):
        f = t * 16 + ii
        plsc.store_scatter(zbuf, [f >> 3, f & 7], zeros)
        return ()

    lax.fori_loop(0, (_RPT + _REM) * _K // 16, zero_body, ())
    pltpu.sync_copy(zbuf.at[pl.ds(0, _RPT)], facc.at[pl.ds(s * _RPT, _RPT)])

    @pl.when(s == _NS - 1)
    def _():
        pltpu.sync_copy(
            zbuf.at[pl.ds(0, _REM)], facc.at[pl.ds(_NS * _RPT, _REM)]
        )

    plsc.subcore_barrier()

    n_chunks = jnp.where(w < _CHUNKS % _NW, _CHUNKS // _NW + 1, _CHUNKS // _NW)

    def issue_inputs(t):
        m = w + t * _NW
        off = (t & 1) * _C
        pltpu.async_copy(
            r_hbm.at[pl.ds(m * _C, _C)], rb.at[pl.ds(off, _C)], sem_r
        )
        pltpu.async_copy(
            fa_hbm.at[pl.ds(m * (_C // 128), _C // 128)],
            idxb.at[pl.ds((t & 1) * (_C // 128), _C // 128)],
            sem_i,
        )

    issue_inputs(0)

    def chunk_body(t, _):
        off = (t & 1) * _C
        ioff = (t & 1) * (_C // 128)

        @pl.when(t > 0)
        def _():
            poff = ((t - 1) & 1) * _C
            pltpu.make_async_copy(
                out_hbm.at[pl.ds(0, _C)], rows.at[pl.ds(poff, _C)], sem_sc
            ).wait()

        @pl.when(t + 1 < n_chunks)
        def _():
            issue_inputs(t + 1)

        pltpu.make_async_copy(
            r_hbm.at[pl.ds(0, _C)], rb.at[pl.ds(off, _C)], sem_r
        ).wait()
        pltpu.make_async_copy(
            fa_hbm.at[pl.ds(0, _C // 128)],
            idxb.at[pl.ds(ioff, _C // 128)],
            sem_i,
        ).wait()

        def group_body(g, _):
            erow = off + g * 16 + ii
            ss = rb[pl.ds(off + g * 16, 16)]
            ssc = jnp.maximum(ss, jnp.float32(1e-37))
            t32 = plsc.bitcast(ssc, jnp.int32)
            t32 = jnp.int32(0x5F3759DF) - (t32 >> 1)
            q = plsc.bitcast(t32, jnp.float32)
            h = ssc * jnp.float32(0.5)
            q = q * (jnp.float32(1.5) - h * q * q)
            q = q * (jnp.float32(1.5) - h * q * q)
            q = q * (jnp.float32(1.5) - h * q * q)
            r = ss * q * jnp.float32(1.0 / _CUTOFF)
            plsc.store_scatter(rows, [erow, jnp.full((16,), 0, jnp.int32)], ones)
            pk = r
            for k in range(1, _K):
                plsc.store_scatter(
                    rows, [erow, jnp.full((16,), k, jnp.int32)], pk
                )
                pk = pk * r
            return ()

        lax.fori_loop(0, _GROUPS, group_body, ())

        for j in range(_C // 128):
            pltpu.async_copy(
                rows.at[pl.ds(off + j * 128, 128)],
                facc.at[idxb.at[ioff + j]],
                sem_sc,
                add=True,
            )
        return ()

    lax.fori_loop(0, n_chunks, chunk_body, ())

    @pl.when(n_chunks > 0)
    def _():
        loff = ((n_chunks - 1) & 1) * _C
        pltpu.make_async_copy(
            out_hbm.at[pl.ds(0, _C)], rows.at[pl.ds(loff, _C)], sem_sc
        ).wait()

    plsc.subcore_barrier()
    r0 = c * _N + s * _RPT
    pltpu.sync_copy(facc.at[pl.ds(s * _RPT, _RPT)], zbuf.at[pl.ds(0, _RPT)])
    pltpu.sync_copy(zbuf.at[pl.ds(0, _RPT)], out_hbm.at[pl.ds(r0, _RPT)])

    @pl.when(s == _NS - 1)
    def _():
        pltpu.sync_copy(
            facc.at[pl.ds(_NS * _RPT, _REM)], zbuf.at[pl.ds(_RPT, _REM)]
        )
        pltpu.sync_copy(
            zbuf.at[pl.ds(_RPT, _REM)],
            out_hbm.at[pl.ds(c * _N + _NS * _RPT, _REM)],
        )


_seg_kernel = functools.partial(
    pl.kernel,
    out_type=jax.ShapeDtypeStruct((_NC * _N, _K), jnp.float32),
    mesh=plsc.VectorSubcoreMesh(core_axis_name="c", subcore_axis_name="s"),
    compiler_params=pltpu.CompilerParams(
        needs_layout_passes=False, use_tc_tiling_on_sc=False
    ),
    scratch_types=[
        pltpu.VMEM((2 * _C,), jnp.float32),
        pltpu.VMEM((2 * (_C // 128), 128), jnp.int32),
        pltpu.VMEM((2 * _C, _K), jnp.float32),
        pltpu.VMEM((_RPT + _REM, _K), jnp.float32),
        pltpu.VMEM_SHARED((_N, _K), jnp.float32),
        pltpu.SemaphoreType.DMA,
        pltpu.SemaphoreType.DMA,
        pltpu.SemaphoreType.DMA,
    ],
)(_seg_body)


def kernel(positions, neighbor_vectors, first_atom):
    n = positions.shape[0]
    ss = jnp.sum(neighbor_vectors * neighbor_vectors, axis=1)
    fa2d = first_atom.reshape(-1, 128)
    parts = _seg_kernel(ss, fa2d)
    features, ftf, cs = _gram_kernel(parts, parts)
    s = cs[0]
    mu = jnp.sum(s) / jnp.float32(n * _K)
    one = jnp.ones((_K,), jnp.float32)
    gram = (
        ftf
        - mu * (jnp.outer(s, one) + jnp.outer(one, s))
        + (n * mu * mu) * jnp.ones((_K, _K), jnp.float32)
    )
    gram = gram + (jnp.float32(1e-6) * jnp.trace(gram)) * jnp.eye(
        _K, dtype=jnp.float32
    )
    a8 = jnp.linalg.cholesky(gram).T
    _, h8, _, _ = _tpu_qdwh.qdwh(a8, is_hermitian=False, max_iterations=10)
    v8, s8 = jax.lax.linalg.eigh(h8, sort_eigenvalues=False)
    order = jnp.argsort(jnp.maximum(s8, 0.0), descending=True)
    return features @ v8[:, order[:3]]
